# baseline (speedup 1.0000x reference)
import functools

import jax
import jax.numpy as jnp
from jax import lax
from jax.experimental import pallas as pl
from jax.experimental.pallas import tpu as pltpu
from jax.experimental.pallas import tpu_sc as plsc

_SEQ = 50
_BT = 8
_DT = 125
_NW = 32
_NTASK = _SEQ * _BT
_TPW = (_NTASK + _NW - 1) // _NW
_QW = 256
_CDT = 8


def _make_gather():
  mesh = plsc.VectorSubcoreMesh(core_axis_name="c", subcore_axis_name="s")

  @functools.partial(
      pl.kernel,
      out_type=jax.ShapeDtypeStruct((_SEQ, _DT, _BT, 8, 128), jnp.float32),
      mesh=mesh,
      compiler_params=pltpu.CompilerParams(use_tc_tiling_on_sc=True,
                                           needs_layout_passes=False),
      scratch_types=[
          pltpu.VMEM((128,), jnp.int32),
          pltpu.VMEM((2, 128, _QW), jnp.float32),
          pltpu.VMEM((_CDT, 8, 128), jnp.float32),
          pltpu.SemaphoreType.DMA,
          pltpu.SemaphoreType.DMA,
      ],
  )
  def gather_kernel(idx_hbm, table_hbm, out_hbm, idx_v, rows_v, z_v,
                    sg0, sg1):
    wid = lax.axis_index("s") * 2 + lax.axis_index("c")
    sems_g = (sg0, sg1)
    bvecs = [lax.iota(jnp.int32, 16) + bq * 16 for bq in range(8)]

    def rows_src(q):
      return table_hbm.at[idx_v, pl.ds(q * _QW, _QW)]

    @pl.loop(0, _TPW)
    def _(j):
      t = wid + _NW * j

      @pl.when(t < _NTASK)
      def _():
        s = t // _BT
        bt = t % _BT
        pltpu.sync_copy(idx_hbm.at[s, pl.ds(bt * 128, 128)], idx_v)
        pltpu.async_copy(rows_src(0), rows_v.at[0], sems_g[0])

        def do_chunk(q, qb, c, nk):
          @plsc.parallel_loop(0, nk, unroll=2)
          def _(k):
            for ds in range(8):
              dcol = c * (_CDT * 8) + k * 8 + ds
              for bq in range(8):
                v = plsc.load_gather(
                    rows_v.at[qb],
                    [bvecs[bq], jnp.broadcast_to(dcol, (16,))])
                z_v[k, ds, pl.ds(bq * 16, 16)] = v

          pltpu.sync_copy(
              z_v.at[pl.ds(0, nk)],
              out_hbm.at[s, pl.ds(q * 32 + c * _CDT, nk), bt])

        for q in range(4):
          qb = q % 2
          pltpu.make_async_copy(rows_src(q), rows_v.at[qb],
                                sems_g[qb]).wait()
          if q < 3:
            pltpu.async_copy(rows_src(q + 1), rows_v.at[1 - qb],
                             sems_g[1 - qb])
          n_dt = 32 if q < 3 else _DT - 96
          n_full = n_dt // _CDT
          tail = n_dt % _CDT

          @pl.loop(0, n_full)
          def _(c):
            do_chunk(q, qb, c, _CDT)
          if tail:
            do_chunk(q, qb, n_full, tail)

  return gather_kernel


@jax.jit
def kernel(indices, embedding):
  batch, seq = indices.shape
  idx_t = indices.astype(jnp.int32).T
  table = jnp.pad(embedding, ((0, 0), (0, 24)))
  z = _make_gather()(idx_t, table)
  return z.transpose(2, 4, 0, 1, 3).reshape(batch, seq, 1000)


# speedup vs baseline: 1.1885x; 1.1885x over previous
import functools

import jax
import jax.numpy as jnp
from jax import lax
from jax.experimental import pallas as pl
from jax.experimental.pallas import tpu as pltpu
from jax.experimental.pallas import tpu_sc as plsc

_SEQ = 50
_BT = 8
_DT = 125
_NW = 32
_NTASK = _SEQ * _BT
_TPW = (_NTASK + _NW - 1) // _NW
_QW = 256
_CDT = 8


def _make_gather():
  mesh = plsc.VectorSubcoreMesh(core_axis_name="c", subcore_axis_name="s")

  @functools.partial(
      pl.kernel,
      out_type=jax.ShapeDtypeStruct((_SEQ, _DT, _BT, 8, 128), jnp.float32),
      mesh=mesh,
      compiler_params=pltpu.CompilerParams(use_tc_tiling_on_sc=True,
                                           needs_layout_passes=False),
      scratch_types=[
          pltpu.VMEM((128,), jnp.int32),
          pltpu.VMEM((2, 128, _QW), jnp.float32),
          pltpu.VMEM((_CDT, 8, 128), jnp.float32),
          pltpu.SemaphoreType.DMA,
          pltpu.SemaphoreType.DMA,
      ],
  )
  def gather_kernel(idx_hbm, table_hbm, out_hbm, idx_v, rows_v, z_v,
                    sg0, sg1):
    wid = lax.axis_index("s") * 2 + lax.axis_index("c")
    sems_g = (sg0, sg1)
    bvecs = [lax.iota(jnp.int32, 16) + bq * 16 for bq in range(8)]

    def rows_src(q):
      return table_hbm.at[idx_v, pl.ds(q * _QW, _QW)]

    @pl.loop(0, _TPW)
    def _(j):
      t = wid + _NW * j

      @pl.when(t < _NTASK)
      def _():
        s = t // _BT
        bt = t % _BT
        pltpu.sync_copy(idx_hbm.at[s, pl.ds(bt * 128, 128)], idx_v)
        pltpu.async_copy(rows_src(0), rows_v.at[0], sems_g[0])

        def do_chunk(q, qb, c, nk):
          @plsc.parallel_loop(0, nk * 8)
          def _(i):
            k = i // 8
            ds = i - k * 8
            dcol = jnp.broadcast_to(c * (_CDT * 8) + i, (16,))
            for bq in range(8):
              v = plsc.load_gather(rows_v.at[qb], [bvecs[bq], dcol])
              z_v[k, ds, pl.ds(bq * 16, 16)] = v

          pltpu.sync_copy(
              z_v.at[pl.ds(0, nk)],
              out_hbm.at[s, pl.ds(q * 32 + c * _CDT, nk), bt])

        for q in range(4):
          qb = q % 2
          pltpu.make_async_copy(rows_src(q), rows_v.at[qb],
                                sems_g[qb]).wait()
          if q < 3:
            pltpu.async_copy(rows_src(q + 1), rows_v.at[1 - qb],
                             sems_g[1 - qb])
          n_dt = 32 if q < 3 else _DT - 96
          n_full = n_dt // _CDT
          tail = n_dt % _CDT

          @pl.loop(0, n_full)
          def _(c):
            do_chunk(q, qb, c, _CDT)
          if tail:
            do_chunk(q, qb, n_full, tail)

  return gather_kernel


@jax.jit
def kernel(indices, embedding):
  batch, seq = indices.shape
  idx_t = indices.astype(jnp.int32).T
  table = jnp.pad(embedding, ((0, 0), (0, 24)))
  z = _make_gather()(idx_t, table)
  return z.transpose(2, 4, 0, 1, 3).reshape(batch, seq, 1000)
